# Initial kernel scaffold; baseline (speedup 1.0000x reference)
import functools

import jax
import jax.numpy as jnp
from jax import lax
from jax.experimental import pallas as pl
from jax.experimental.pallas import tpu as pltpu
from jax.experimental.pallas import tpu_sc as plsc

D_NUM = 64
D_EMB = 128
D_OUT = D_NUM + 3 * D_EMB
CHUNK = 128


@functools.lru_cache(maxsize=None)
def _make_kernel(N: int):
    info = plsc.get_sparse_core_info()
    NC, NS = info.num_cores, info.num_subcores
    NW = NC * NS
    assert N % (NW * CHUNK) == 0
    per_w = N // NW
    n_iter = per_w // CHUNK

    mesh = plsc.VectorSubcoreMesh(core_axis_name="c", subcore_axis_name="s")

    @functools.partial(
        pl.kernel,
        mesh=mesh,
        out_type=jax.ShapeDtypeStruct((N, D_OUT), jnp.float32),
        scratch_types=[
            pltpu.VMEM((CHUNK,), jnp.int32),
            pltpu.VMEM((CHUNK,), jnp.int32),
            pltpu.VMEM((CHUNK,), jnp.int32),
            pltpu.VMEM((CHUNK, D_NUM), jnp.float32),
            pltpu.VMEM((CHUNK, D_EMB), jnp.float32),
            pltpu.VMEM((CHUNK, D_EMB), jnp.float32),
            pltpu.VMEM((CHUNK, D_EMB), jnp.float32),
            pltpu.SemaphoreType.DMA,
        ],
    )
    def k(numeric, wp_id, gl_id, ac_id, w_wp, w_gl, w_ac, out,
          wi_v, gi_v, ai_v, num_v, wp_v, gl_v, ac_v, sem):
        wid = lax.axis_index("s") * NC + lax.axis_index("c")
        w_base = wid * per_w

        def body(i, carry):
            base = w_base + i * CHUNK
            pltpu.sync_copy(wp_id.at[pl.ds(base, CHUNK)], wi_v)
            pltpu.sync_copy(gl_id.at[pl.ds(base, CHUNK)], gi_v)
            pltpu.sync_copy(ac_id.at[pl.ds(base, CHUNK)], ai_v)
            cn = pltpu.async_copy(
                numeric.at[pl.ds(base, CHUNK), :], num_v, sem)
            cw = pltpu.async_copy(w_wp.at[wi_v], wp_v, sem)
            cg = pltpu.async_copy(w_gl.at[gi_v], gl_v, sem)
            ca = pltpu.async_copy(w_ac.at[ai_v], ac_v, sem)
            cn.wait()
            cw.wait()
            cg.wait()
            ca.wait()
            pltpu.sync_copy(
                num_v, out.at[pl.ds(base, CHUNK), pl.ds(0, D_NUM)])
            pltpu.sync_copy(
                wp_v, out.at[pl.ds(base, CHUNK), pl.ds(D_NUM, D_EMB)])
            pltpu.sync_copy(
                gl_v, out.at[pl.ds(base, CHUNK), pl.ds(D_NUM + D_EMB, D_EMB)])
            pltpu.sync_copy(
                ac_v,
                out.at[pl.ds(base, CHUNK), pl.ds(D_NUM + 2 * D_EMB, D_EMB)])
            return carry

        lax.fori_loop(0, n_iter, body, 0)

    return k


def kernel(numeric, waypoint_id, final_goal_id, action_id, W_wp, W_gl, W_ac):
    B, L, d_num = numeric.shape
    N = B * L
    num2 = numeric.reshape(N, d_num)
    wi = waypoint_id.reshape(N).astype(jnp.int32)
    gi = final_goal_id.reshape(N).astype(jnp.int32)
    ai = action_id.reshape(N).astype(jnp.int32)
    out = _make_kernel(N)(num2, wi, gi, ai, W_wp, W_gl, W_ac)
    return out.reshape(B, L, D_OUT)


# speedup vs baseline: 2.8405x; 2.8405x over previous
import functools

import jax
import jax.numpy as jnp
from jax import lax
from jax.experimental import pallas as pl
from jax.experimental.pallas import tpu as pltpu
from jax.experimental.pallas import tpu_sc as plsc

D_NUM = 64
D_EMB = 128
D_OUT = D_NUM + 3 * D_EMB
CHUNK = 128


@functools.lru_cache(maxsize=None)
def _make_kernel(N: int):
    info = plsc.get_sparse_core_info()
    NC, NS = info.num_cores, info.num_subcores
    NW = NC * NS
    assert N % (NW * CHUNK) == 0
    per_w = N // NW
    n_iter = per_w // CHUNK

    mesh = plsc.VectorSubcoreMesh(core_axis_name="c", subcore_axis_name="s")

    @functools.partial(
        pl.kernel,
        mesh=mesh,
        out_type=jax.ShapeDtypeStruct((N, D_OUT), jnp.float32),
        scratch_types=[
            pltpu.VMEM((CHUNK,), jnp.int32),
            pltpu.VMEM((CHUNK,), jnp.int32),
            pltpu.VMEM((CHUNK,), jnp.int32),
            pltpu.VMEM((CHUNK, D_NUM), jnp.float32),
            pltpu.VMEM((CHUNK, D_EMB), jnp.float32),
            pltpu.VMEM((CHUNK, D_EMB), jnp.float32),
            pltpu.VMEM((CHUNK, D_EMB), jnp.float32),
            pltpu.SemaphoreType.DMA,
        ],
        compiler_params=pltpu.CompilerParams(use_tc_tiling_on_sc=False),
    )
    def k(numeric, wp_id, gl_id, ac_id, w_wp, w_gl, w_ac, out,
          wi_v, gi_v, ai_v, num_v, wp_v, gl_v, ac_v, sem):
        wid = lax.axis_index("s") * NC + lax.axis_index("c")
        w_base = wid * per_w

        def body(i, carry):
            base = w_base + i * CHUNK
            pltpu.sync_copy(wp_id.at[pl.ds(base, CHUNK)], wi_v)
            pltpu.sync_copy(gl_id.at[pl.ds(base, CHUNK)], gi_v)
            pltpu.sync_copy(ac_id.at[pl.ds(base, CHUNK)], ai_v)
            cn = pltpu.async_copy(
                numeric.at[pl.ds(base, CHUNK), :], num_v, sem)
            cw = pltpu.async_copy(w_wp.at[wi_v], wp_v, sem)
            cg = pltpu.async_copy(w_gl.at[gi_v], gl_v, sem)
            ca = pltpu.async_copy(w_ac.at[ai_v], ac_v, sem)
            cn.wait()
            cw.wait()
            cg.wait()
            ca.wait()
            pltpu.sync_copy(
                num_v, out.at[pl.ds(base, CHUNK), pl.ds(0, D_NUM)])
            pltpu.sync_copy(
                wp_v, out.at[pl.ds(base, CHUNK), pl.ds(D_NUM, D_EMB)])
            pltpu.sync_copy(
                gl_v, out.at[pl.ds(base, CHUNK), pl.ds(D_NUM + D_EMB, D_EMB)])
            pltpu.sync_copy(
                ac_v,
                out.at[pl.ds(base, CHUNK), pl.ds(D_NUM + 2 * D_EMB, D_EMB)])
            return carry

        lax.fori_loop(0, n_iter, body, 0)

    return k


def kernel(numeric, waypoint_id, final_goal_id, action_id, W_wp, W_gl, W_ac):
    B, L, d_num = numeric.shape
    N = B * L
    num2 = numeric.reshape(N, d_num)
    wi = waypoint_id.reshape(N).astype(jnp.int32)
    gi = final_goal_id.reshape(N).astype(jnp.int32)
    ai = action_id.reshape(N).astype(jnp.int32)
    out = _make_kernel(N)(num2, wi, gi, ai, W_wp, W_gl, W_ac)
    return out.reshape(B, L, D_OUT)


# speedup vs baseline: 3.0621x; 1.0780x over previous
import functools

import jax
import jax.numpy as jnp
from jax import lax
from jax.experimental import pallas as pl
from jax.experimental.pallas import tpu as pltpu
from jax.experimental.pallas import tpu_sc as plsc

D_NUM = 64
D_EMB = 128
D_OUT = D_NUM + 3 * D_EMB
CHUNK = 128


@functools.lru_cache(maxsize=None)
def _make_kernel(N: int):
    info = plsc.get_sparse_core_info()
    NC, NS = info.num_cores, info.num_subcores
    NW = NC * NS
    assert N % (NW * CHUNK * 2) == 0
    per_w = N // NW
    n_iter = per_w // CHUNK

    mesh = plsc.VectorSubcoreMesh(core_axis_name="c", subcore_axis_name="s")

    slot_t = [
        pltpu.VMEM((CHUNK,), jnp.int32),
        pltpu.VMEM((CHUNK,), jnp.int32),
        pltpu.VMEM((CHUNK,), jnp.int32),
        pltpu.VMEM((CHUNK, D_NUM), jnp.float32),
        pltpu.VMEM((CHUNK, D_EMB), jnp.float32),
        pltpu.VMEM((CHUNK, D_EMB), jnp.float32),
        pltpu.VMEM((CHUNK, D_EMB), jnp.float32),
        pltpu.SemaphoreType.DMA,
        pltpu.SemaphoreType.DMA,
    ]

    @functools.partial(
        pl.kernel,
        mesh=mesh,
        out_type=jax.ShapeDtypeStruct((N, D_OUT), jnp.float32),
        scratch_types=slot_t + slot_t,
        compiler_params=pltpu.CompilerParams(use_tc_tiling_on_sc=False),
    )
    def k(numeric, wp_id, gl_id, ac_id, w_wp, w_gl, w_ac, out, *scratch):
        slots = (scratch[:9], scratch[9:])
        wid = lax.axis_index("s") * NC + lax.axis_index("c")
        w_base = wid * per_w

        def issue_in(s, base):
            wi, gi, ai, num, wp, gl, ac, semg, _ = slots[s]
            pltpu.sync_copy(wp_id.at[pl.ds(base, CHUNK)], wi)
            pltpu.sync_copy(gl_id.at[pl.ds(base, CHUNK)], gi)
            pltpu.sync_copy(ac_id.at[pl.ds(base, CHUNK)], ai)
            pltpu.async_copy(numeric.at[pl.ds(base, CHUNK), :], num, semg)
            pltpu.async_copy(w_wp.at[wi], wp, semg)
            pltpu.async_copy(w_gl.at[gi], gl, semg)
            pltpu.async_copy(w_ac.at[ai], ac, semg)

        def wait_in(s, base):
            wi, gi, ai, num, wp, gl, ac, semg, _ = slots[s]
            pltpu.make_async_copy(
                numeric.at[pl.ds(base, CHUNK), :], num, semg).wait()
            pltpu.make_async_copy(w_wp.at[wi], wp, semg).wait()
            pltpu.make_async_copy(w_gl.at[gi], gl, semg).wait()
            pltpu.make_async_copy(w_ac.at[ai], ac, semg).wait()

        def issue_out(s, base):
            _, _, _, num, wp, gl, ac, _, semw = slots[s]
            rows = pl.ds(base, CHUNK)
            pltpu.async_copy(num, out.at[rows, pl.ds(0, D_NUM)], semw)
            pltpu.async_copy(wp, out.at[rows, pl.ds(D_NUM, D_EMB)], semw)
            pltpu.async_copy(
                gl, out.at[rows, pl.ds(D_NUM + D_EMB, D_EMB)], semw)
            pltpu.async_copy(
                ac, out.at[rows, pl.ds(D_NUM + 2 * D_EMB, D_EMB)], semw)

        def wait_out(s, base):
            _, _, _, num, wp, gl, ac, _, semw = slots[s]
            rows = pl.ds(base, CHUNK)
            pltpu.make_async_copy(
                num, out.at[rows, pl.ds(0, D_NUM)], semw).wait()
            pltpu.make_async_copy(
                wp, out.at[rows, pl.ds(D_NUM, D_EMB)], semw).wait()
            pltpu.make_async_copy(
                gl, out.at[rows, pl.ds(D_NUM + D_EMB, D_EMB)], semw).wait()
            pltpu.make_async_copy(
                ac, out.at[rows, pl.ds(D_NUM + 2 * D_EMB, D_EMB)], semw).wait()

        issue_in(0, w_base)
        issue_in(1, w_base + CHUNK)

        def body(j, carry):
            for s in (0, 1):
                i = 2 * j + s
                base = w_base + i * CHUNK
                wait_in(s, base)
                issue_out(s, base)
                wait_out(s, base)

                @pl.when(i + 2 < n_iter)
                def _():
                    issue_in(s, base + 2 * CHUNK)
            return carry

        lax.fori_loop(0, n_iter // 2, body, 0)

    return k


def kernel(numeric, waypoint_id, final_goal_id, action_id, W_wp, W_gl, W_ac):
    B, L, d_num = numeric.shape
    N = B * L
    num2 = numeric.reshape(N, d_num)
    wi = waypoint_id.reshape(N).astype(jnp.int32)
    gi = final_goal_id.reshape(N).astype(jnp.int32)
    ai = action_id.reshape(N).astype(jnp.int32)
    out = _make_kernel(N)(num2, wi, gi, ai, W_wp, W_gl, W_ac)
    return out.reshape(B, L, D_OUT)
